# baseline (speedup 1.0000x reference)
import functools

import jax
import jax.numpy as jnp
from jax import lax
from jax.experimental import pallas as pl
from jax.experimental.pallas import tpu as pltpu
from jax.experimental.pallas import tpu_sc as plsc

NC = 2
NS = 16
NW = NC * NS

_TBLK = 512


def _tc_relayout(table_t):
    embed, vocab = table_t.shape
    wide = 2 * embed

    def body(x_ref, o_ref):
        y = x_ref[...].T
        o_ref[...] = jnp.concatenate([y, y], axis=1)

    return pl.pallas_call(
        body,
        grid=(vocab // _TBLK,),
        in_specs=[pl.BlockSpec((embed, _TBLK), lambda j: (0, j))],
        out_specs=pl.BlockSpec((_TBLK, wide), lambda j: (j, 0)),
        out_shape=jax.ShapeDtypeStruct((vocab, wide), jnp.float32),
    )(table_t)


def _sc_partial_sums(text, offsets, table_wide, *, total, batch, embed):
    n_per_tile = total // NW
    n_chunks = n_per_tile // 128
    wide = 2 * embed

    mesh = plsc.VectorSubcoreMesh(core_axis_name="c", subcore_axis_name="s")

    @functools.partial(
        pl.kernel,
        out_type=jax.ShapeDtypeStruct((NC, batch, wide), jnp.float32),
        mesh=mesh,
        compiler_params=pltpu.CompilerParams(needs_layout_passes=False),
        scratch_types=[
            pltpu.VMEM((n_per_tile,), jnp.int32),
            pltpu.VMEM((batch,), jnp.int32),
            pltpu.VMEM((n_per_tile,), jnp.int32),
            pltpu.VMEM((n_chunks, 128), jnp.int32),
            pltpu.VMEM((128, wide), jnp.float32),
            pltpu.VMEM((16, wide), jnp.float32),
            pltpu.VMEM((16,), jnp.int32),
            pltpu.VMEM_SHARED((batch + NS * 16, wide), jnp.float32),
            pltpu.SemaphoreType.DMA,
        ],
    )
    def k(text_hbm, off_hbm, table_hbm, psums_hbm,
          text_v, off_v, marks_v, seg_v, rows_v, red_v, fidx_v, acc_sh, sem):
        cid = lax.axis_index("c")
        sid = lax.axis_index("s")
        wid = cid * NS + sid
        t0 = wid * n_per_tile
        t1 = t0 + n_per_tile
        ncol = embed // 16

        pltpu.sync_copy(off_hbm, off_v)
        pltpu.sync_copy(text_hbm.at[pl.ds(t0, n_per_tile)], text_v)

        zrow = jnp.zeros((16,), jnp.float32)

        def _zero_rows(t, _):
            for c in range(wide // 16):
                rows_v[t, pl.ds(c * 16, 16)] = zrow
            return 0

        lax.fori_loop(0, 128, _zero_rows, 0)
        rows_per_tile = batch // NS
        for h in range(rows_per_tile // 128):
            pltpu.sync_copy(rows_v, acc_sh.at[pl.ds(sid * rows_per_tile + h * 128, 128)])

        zrow_i = jnp.zeros((16,), jnp.int32)

        def _zero_marks(i, _):
            marks_v[pl.ds(i * 16, 16)] = zrow_i
            return 0

        lax.fori_loop(0, n_per_tile // 16, _zero_marks, 0)

        ones_i = jnp.ones((16,), jnp.int32)

        def _scan_offs(r, base_vec):
            o = off_v[pl.ds(r * 16, 16)]
            in_rng = jnp.logical_and(o >= t0, o < t1)
            p = o - t0
            plsc.addupdate_scatter(marks_v, [p], ones_i, mask=in_rng)
            return base_vec + jnp.where(o < t0, 1, 0)

        base_vec = lax.fori_loop(0, batch // 16, _scan_offs, zrow_i)
        base = jnp.sum(base_vec)

        def _cumsum_row(r, carry):
            for c in range(8):
                x = marks_v[pl.ds(r * 128 + c * 16, 16)]
                s = plsc.cumsum(x) + carry
                seg_v[r, pl.ds(c * 16, 16)] = s
                carry = carry + jnp.sum(x)
            return carry

        lax.fori_loop(0, n_chunks, _cumsum_row, base - 1)

        plsc.subcore_barrier()

        lane = lax.iota(jnp.int32, 16)
        trash_vec = batch + sid * 16 + lane
        fidx_v[...] = trash_vec

        def _chunk(r, _):
            tb = pl.multiple_of(r * 128, 128)
            pltpu.async_copy(
                table_hbm.at[text_v.at[pl.ds(tb, 128)]], rows_v, sem).wait()
            s_first = seg_v[r, pl.ds(0, 16)][0]
            s_last = seg_v[r, pl.ds(112, 16)][15]
            uniform = s_first == s_last
            j = lax.bitwise_and(r, 15)

            @pl.when(uniform)
            def _():
                def _red(t, acc):
                    return tuple(
                        acc[c] + rows_v[t, pl.ds(c * 16, 16)]
                        for c in range(ncol))

                acc = lax.fori_loop(
                    0, 128, _red,
                    tuple(jnp.zeros((16,), jnp.float32)
                          for _ in range(ncol)))
                for c in range(ncol):
                    red_v[j, pl.ds(c * 16, 16)] = acc[c]
                fidx_v[...] = jnp.where(lane == j, s_first, fidx_v[...])

            @pl.when(jnp.logical_not(uniform))
            def _():
                pltpu.sync_copy(rows_v, acc_sh.at[seg_v.at[r]], add=True)

            @pl.when(j == 15)
            def _():
                pltpu.sync_copy(red_v, acc_sh.at[fidx_v], add=True)
                fidx_v[...] = trash_vec

            return 0

        lax.fori_loop(0, n_chunks, _chunk, 0)
        if n_chunks % 16 != 0:
            pltpu.sync_copy(red_v, acc_sh.at[fidx_v], add=True)

        plsc.subcore_barrier()

        for h in range(rows_per_tile // 128):
            row0 = pl.multiple_of(sid * rows_per_tile + h * 128, 128)
            pltpu.sync_copy(acc_sh.at[pl.ds(row0, 128)], rows_v)
            pltpu.sync_copy(rows_v, psums_hbm.at[cid, pl.ds(row0, 128)])

    return k(text, offsets, table_wide)


def _tc_head(psums, off_col, ends_col, W, b_row):
    batch, wide = psums.shape[1], psums.shape[2]
    embed = wide // 2
    nclass = W.shape[0]

    def body(p_ref, off_ref, ends_ref, w_ref, b_ref, o_ref):
        s = p_ref[0, :, :embed] + p_ref[1, :, :embed]
        cnt = (ends_ref[...] - off_ref[...]).astype(jnp.float32)
        inv = 1.0 / jnp.maximum(cnt, 1.0)
        emb = s * inv
        o_ref[...] = lax.dot_general(
            emb, w_ref[...],
            dimension_numbers=(((1,), (1,)), ((), ())),
            preferred_element_type=jnp.float32) + b_ref[...]

    return pl.pallas_call(
        body,
        in_specs=[
            pl.BlockSpec((NC, batch, wide), lambda: (0, 0, 0)),
            pl.BlockSpec((batch, 1), lambda: (0, 0)),
            pl.BlockSpec((batch, 1), lambda: (0, 0)),
            pl.BlockSpec((nclass, embed), lambda: (0, 0)),
            pl.BlockSpec((1, nclass), lambda: (0, 0)),
        ],
        out_specs=pl.BlockSpec((batch, nclass), lambda: (0, 0)),
        out_shape=jax.ShapeDtypeStruct((batch, nclass), jnp.float32),
    )(psums, off_col, ends_col, W, b_row)


def kernel(text, offsets, table, W, b):
    total = text.shape[0]
    batch = offsets.shape[0]
    vocab, embed = table.shape

    table_wide = _tc_relayout(table.T)
    psums = _sc_partial_sums(text, offsets, table_wide,
                             total=total, batch=batch, embed=embed)

    ends = jnp.concatenate(
        [offsets[1:], jnp.array([total], dtype=offsets.dtype)])
    return _tc_head(psums, offsets.reshape(batch, 1),
                    ends.reshape(batch, 1), W, b.reshape(1, -1))


# speedup vs baseline: 2.1538x; 2.1538x over previous
import functools

import jax
import jax.numpy as jnp
from jax import lax
from jax.experimental import pallas as pl
from jax.experimental.pallas import tpu as pltpu
from jax.experimental.pallas import tpu_sc as plsc

NC = 2
NS = 16
NW = NC * NS

_TBLK = 2048


def _tc_relayout(table_t):
    embed, vocab = table_t.shape
    wide = 2 * embed
    ident = jnp.eye(embed, dtype=jnp.float32)

    def body(x_ref, i_ref, o_ref):
        y = lax.dot_general(
            x_ref[...], i_ref[...],
            dimension_numbers=(((0,), (0,)), ((), ())),
            preferred_element_type=jnp.float32)
        o_ref[...] = jnp.concatenate([y, y], axis=1)

    return pl.pallas_call(
        body,
        grid=(vocab // _TBLK,),
        in_specs=[pl.BlockSpec((embed, _TBLK), lambda j: (0, j)),
                  pl.BlockSpec((embed, embed), lambda j: (0, 0))],
        out_specs=pl.BlockSpec((_TBLK, wide), lambda j: (j, 0)),
        out_shape=jax.ShapeDtypeStruct((vocab, wide), jnp.float32),
    )(table_t, ident)


def _sc_partial_sums(text, offsets, table_wide, *, total, batch, embed):
    n_per_tile = total // NW
    n_chunks = n_per_tile // 128
    wide = 2 * embed

    mesh = plsc.VectorSubcoreMesh(core_axis_name="c", subcore_axis_name="s")

    @functools.partial(
        pl.kernel,
        out_type=jax.ShapeDtypeStruct((NC, batch, wide), jnp.float32),
        mesh=mesh,
        compiler_params=pltpu.CompilerParams(needs_layout_passes=False),
        scratch_types=[
            pltpu.VMEM((n_per_tile,), jnp.int32),
            pltpu.VMEM((batch,), jnp.int32),
            pltpu.VMEM((n_per_tile,), jnp.int32),
            pltpu.VMEM((n_chunks, 128), jnp.int32),
            pltpu.VMEM((128, wide), jnp.float32),
            pltpu.VMEM((16, wide), jnp.float32),
            pltpu.VMEM((16,), jnp.int32),
            pltpu.VMEM_SHARED((batch + NS * 16, wide), jnp.float32),
            pltpu.SemaphoreType.DMA,
        ],
    )
    def k(text_hbm, off_hbm, table_hbm, psums_hbm,
          text_v, off_v, marks_v, seg_v, rows_v, red_v, fidx_v, acc_sh, sem):
        cid = lax.axis_index("c")
        sid = lax.axis_index("s")
        wid = cid * NS + sid
        t0 = wid * n_per_tile
        t1 = t0 + n_per_tile
        ncol = embed // 16

        pltpu.sync_copy(off_hbm, off_v)
        pltpu.sync_copy(text_hbm.at[pl.ds(t0, n_per_tile)], text_v)

        zrow = jnp.zeros((16,), jnp.float32)

        def _zero_rows(t, _):
            for c in range(wide // 16):
                rows_v[t, pl.ds(c * 16, 16)] = zrow
            return 0

        lax.fori_loop(0, 128, _zero_rows, 0)
        rows_per_tile = batch // NS
        for h in range(rows_per_tile // 128):
            pltpu.sync_copy(rows_v, acc_sh.at[pl.ds(sid * rows_per_tile + h * 128, 128)])

        zrow_i = jnp.zeros((16,), jnp.int32)

        def _zero_marks(i, _):
            marks_v[pl.ds(i * 16, 16)] = zrow_i
            return 0

        lax.fori_loop(0, n_per_tile // 16, _zero_marks, 0)

        ones_i = jnp.ones((16,), jnp.int32)

        def _scan_offs(r, base_vec):
            o = off_v[pl.ds(r * 16, 16)]
            in_rng = jnp.logical_and(o >= t0, o < t1)
            p = o - t0
            plsc.addupdate_scatter(marks_v, [p], ones_i, mask=in_rng)
            return base_vec + jnp.where(o < t0, 1, 0)

        base_vec = lax.fori_loop(0, batch // 16, _scan_offs, zrow_i)
        base = jnp.sum(base_vec)

        def _cumsum_row(r, carry):
            for c in range(8):
                x = marks_v[pl.ds(r * 128 + c * 16, 16)]
                s = plsc.cumsum(x) + carry
                seg_v[r, pl.ds(c * 16, 16)] = s
                carry = carry + jnp.sum(x)
            return carry

        lax.fori_loop(0, n_chunks, _cumsum_row, base - 1)

        plsc.subcore_barrier()

        lane = lax.iota(jnp.int32, 16)
        trash_vec = batch + sid * 16 + lane
        fidx_v[...] = trash_vec

        def _chunk(r, _):
            tb = pl.multiple_of(r * 128, 128)
            pltpu.async_copy(
                table_hbm.at[text_v.at[pl.ds(tb, 128)]], rows_v, sem).wait()
            s_first = seg_v[r, pl.ds(0, 16)][0]
            s_last = seg_v[r, pl.ds(112, 16)][15]
            uniform = s_first == s_last
            j = lax.bitwise_and(r, 15)

            @pl.when(uniform)
            def _():
                def _red(t, acc):
                    return tuple(
                        acc[c] + rows_v[t, pl.ds(c * 16, 16)]
                        for c in range(ncol))

                acc = lax.fori_loop(
                    0, 128, _red,
                    tuple(jnp.zeros((16,), jnp.float32)
                          for _ in range(ncol)))
                for c in range(ncol):
                    red_v[j, pl.ds(c * 16, 16)] = acc[c]
                fidx_v[...] = jnp.where(lane == j, s_first, fidx_v[...])

            @pl.when(jnp.logical_not(uniform))
            def _():
                pltpu.sync_copy(rows_v, acc_sh.at[seg_v.at[r]], add=True)

            @pl.when(j == 15)
            def _():
                pltpu.sync_copy(red_v, acc_sh.at[fidx_v], add=True)
                fidx_v[...] = trash_vec

            return 0

        lax.fori_loop(0, n_chunks, _chunk, 0)
        if n_chunks % 16 != 0:
            pltpu.sync_copy(red_v, acc_sh.at[fidx_v], add=True)

        plsc.subcore_barrier()

        for h in range(rows_per_tile // 128):
            row0 = pl.multiple_of(sid * rows_per_tile + h * 128, 128)
            pltpu.sync_copy(acc_sh.at[pl.ds(row0, 128)], rows_v)
            pltpu.sync_copy(rows_v, psums_hbm.at[cid, pl.ds(row0, 128)])

    return k(text, offsets, table_wide)


def _tc_head(psums, off_col, ends_col, W, b_row):
    batch, wide = psums.shape[1], psums.shape[2]
    embed = wide // 2
    nclass = W.shape[0]

    def body(p_ref, off_ref, ends_ref, w_ref, b_ref, o_ref):
        s = p_ref[0, :, :embed] + p_ref[1, :, :embed]
        cnt = (ends_ref[...] - off_ref[...]).astype(jnp.float32)
        inv = 1.0 / jnp.maximum(cnt, 1.0)
        emb = s * inv
        o_ref[...] = lax.dot_general(
            emb, w_ref[...],
            dimension_numbers=(((1,), (1,)), ((), ())),
            preferred_element_type=jnp.float32) + b_ref[...]

    return pl.pallas_call(
        body,
        in_specs=[
            pl.BlockSpec((NC, batch, wide), lambda: (0, 0, 0)),
            pl.BlockSpec((batch, 1), lambda: (0, 0)),
            pl.BlockSpec((batch, 1), lambda: (0, 0)),
            pl.BlockSpec((nclass, embed), lambda: (0, 0)),
            pl.BlockSpec((1, nclass), lambda: (0, 0)),
        ],
        out_specs=pl.BlockSpec((batch, nclass), lambda: (0, 0)),
        out_shape=jax.ShapeDtypeStruct((batch, nclass), jnp.float32),
    )(psums, off_col, ends_col, W, b_row)


def kernel(text, offsets, table, W, b):
    total = text.shape[0]
    batch = offsets.shape[0]
    vocab, embed = table.shape

    table_wide = _tc_relayout(table.T)
    psums = _sc_partial_sums(text, offsets, table_wide,
                             total=total, batch=batch, embed=embed)

    ends = jnp.concatenate(
        [offsets[1:], jnp.array([total], dtype=offsets.dtype)])
    return _tc_head(psums, offsets.reshape(batch, 1),
                    ends.reshape(batch, 1), W, b.reshape(1, -1))
